# baseline (speedup 1.0000x reference)
import functools

import jax
import jax.numpy as jnp
import numpy as np
from jax.experimental import pallas as pl
from jax.experimental.pallas import tpu as pltpu

_BN_EPS = 1e-5
_POOLK = (4, 8, 16, 32)


def _pool1d(n, k):
    out = n // k
    m = np.zeros((out, n), np.float32)
    for i in range(out):
        m[i, i * k:(i + 1) * k] = 1.0 / k
    return m


def _up1d(n_out, n_in):
    m = np.zeros((n_out, n_in), np.float32)
    for i in range(n_out):
        p = i * (n_in - 1) / (n_out - 1) if n_out > 1 else 0.0
        lo = int(np.floor(p))
        hi = min(lo + 1, n_in - 1)
        f = p - lo
        m[i, lo] += 1.0 - f
        m[i, hi] += f
    return m


def _spp_fused_kernel(x_ref, a_ref, u_ref, w1_ref, b1_ref, w9_ref, bo_ref,
                      o_ref, z_ref, *, H, W):
    C = o_ref.shape[-2]
    WP = W + 2
    n_rows = H * WP

    x = jnp.transpose(x_ref[0].astype(jnp.bfloat16))

    feats = []
    for i in range(4):
        pooled = jnp.dot(a_ref[i], x, preferred_element_type=jnp.float32)
        g = jnp.maximum(
            jnp.dot(pooled.astype(jnp.bfloat16), w1_ref[i],
                    preferred_element_type=jnp.float32) + b1_ref[i], 0.0)
        up = jnp.dot(u_ref[i], g.astype(jnp.bfloat16),
                     preferred_element_type=jnp.float32)
        feats.append(up.astype(jnp.bfloat16))
    feats.append(x)
    z = jnp.concatenate(feats, axis=-1)

    z_ref[...] = jnp.zeros(z_ref.shape, z_ref.dtype)
    for r in range(H):
        z_ref[pl.ds((r + 1) * WP + 1, W), :] = z[r * W:(r + 1) * W, :]

    acc = jnp.zeros((n_rows, C), jnp.float32)
    for t in range(9):
        off = (t // 3) * WP + (t % 3)
        acc = acc + jnp.dot(z_ref[pl.ds(off, n_rows), :], w9_ref[t],
                            preferred_element_type=jnp.float32)
    out = jnp.maximum(acc + bo_ref[...], 0.0)

    clean = jnp.concatenate([out[r * WP:r * WP + W] for r in range(H)], axis=0)
    o_ref[0] = jnp.transpose(clean)


def kernel(x, branch0_w, branch0_b, branch0_gamma, branch0_beta, branch0_mean,
           branch0_var, branch1_w, branch1_b, branch1_gamma, branch1_beta,
           branch1_mean, branch1_var, branch2_w, branch2_b, branch2_gamma,
           branch2_beta, branch2_mean, branch2_var, branch3_w, branch3_b,
           branch3_gamma, branch3_beta, branch3_mean, branch3_var,
           out_w, out_b, out_gamma, out_beta, out_mean, out_var):
    B, C, H, W = x.shape
    HW = H * W
    C5 = 5 * C
    P = max(8, -(-((H // 4) * (W // 4)) // 8) * 8)

    a_np, u_np = [], []
    for k in _POOLK:
        a = np.kron(_pool1d(H, k), _pool1d(W, k))
        u = np.kron(_up1d(H, H // k), _up1d(W, W // k))
        hw = a.shape[0]
        a_np.append(np.pad(a, ((0, P - hw), (0, 0))))
        u_np.append(np.pad(u, ((0, 0), (0, P - hw))))
    a_all = jnp.asarray(np.stack(a_np), jnp.bfloat16)
    u_all = jnp.asarray(np.stack(u_np), jnp.bfloat16)

    w1_l, b1_l = [], []
    for w, b, gamma, beta, mean, var in (
            (branch0_w, branch0_b, branch0_gamma, branch0_beta, branch0_mean, branch0_var),
            (branch1_w, branch1_b, branch1_gamma, branch1_beta, branch1_mean, branch1_var),
            (branch2_w, branch2_b, branch2_gamma, branch2_beta, branch2_mean, branch2_var),
            (branch3_w, branch3_b, branch3_gamma, branch3_beta, branch3_mean, branch3_var)):
        s = gamma * jax.lax.rsqrt(var + _BN_EPS)
        w1_l.append((w * s[:, None]).T)
        b1_l.append(s * (b - mean) + beta)
    w1 = jnp.stack(w1_l).astype(jnp.bfloat16)
    b1 = jnp.stack(b1_l).reshape(4, 1, C)

    so = out_gamma * jax.lax.rsqrt(out_var + _BN_EPS)
    wf = out_w * so[:, None, None, None]
    w9 = jnp.transpose(wf, (2, 3, 1, 0)).reshape(9, C5, C).astype(jnp.bfloat16)
    bo = (so * (out_b - out_mean) + out_beta).reshape(1, C)

    x_cl = x.reshape(B, C, HW)

    WP = W + 2
    zrows = (H + 3) * WP
    flops = B * (2 * 4 * (P * HW * C + P * C * C + HW * P * C)
                 + 2 * 9 * H * WP * C5 * C)
    bytes_accessed = 4 * (B * HW * C * 2 + C5 * C * 9) + 2 * (4 * P * HW * 2)
    out_flat = pl.pallas_call(
        functools.partial(_spp_fused_kernel, H=H, W=W),
        out_shape=jax.ShapeDtypeStruct((B, C, HW), jnp.float32),
        grid=(B,),
        in_specs=[
            pl.BlockSpec((1, C, HW), lambda b: (b, 0, 0)),
            pl.BlockSpec((4, P, HW), lambda b: (0, 0, 0)),
            pl.BlockSpec((4, HW, P), lambda b: (0, 0, 0)),
            pl.BlockSpec((4, C, C), lambda b: (0, 0, 0)),
            pl.BlockSpec((4, 1, C), lambda b: (0, 0, 0)),
            pl.BlockSpec((9, C5, C), lambda b: (0, 0, 0)),
            pl.BlockSpec((1, C), lambda b: (0, 0)),
        ],
        out_specs=pl.BlockSpec((1, C, HW), lambda b: (b, 0, 0)),
        scratch_shapes=[pltpu.VMEM((zrows, C5), jnp.bfloat16)],
        compiler_params=pltpu.CompilerParams(
            dimension_semantics=("parallel",),
            vmem_limit_bytes=48 * 1024 * 1024,
        ),
        cost_estimate=pl.CostEstimate(flops=flops, transcendentals=0,
                                      bytes_accessed=bytes_accessed),
    )(x_cl, a_all, u_all, w1, b1, w9, bo)

    return out_flat.reshape(B, C, H, W)


# speedup vs baseline: 1.6571x; 1.6571x over previous
import functools

import jax
import jax.numpy as jnp
import numpy as np
from jax.experimental import pallas as pl
from jax.experimental.pallas import tpu as pltpu

_BN_EPS = 1e-5
_POOLK = (4, 8, 16, 32)


def _pool1d(n, k):
    out = n // k
    m = np.zeros((out, n), np.float32)
    for i in range(out):
        m[i, i * k:(i + 1) * k] = 1.0 / k
    return m


def _up1d(n_out, n_in):
    m = np.zeros((n_out, n_in), np.float32)
    for i in range(n_out):
        p = i * (n_in - 1) / (n_out - 1) if n_out > 1 else 0.0
        lo = int(np.floor(p))
        hi = min(lo + 1, n_in - 1)
        f = p - lo
        m[i, lo] += 1.0 - f
        m[i, hi] += f
    return m


def _build_operators(H, W, WP):
    HW = H * W
    hw_list = [(H // k) * (W // k) for k in _POOLK]
    p_list = [max(8, -(-hw // 8) * 8) for hw in hw_list]
    p_bases = np.cumsum([0] + p_list)[:-1].tolist()
    psum = sum(p_list)

    a_stack = np.zeros((psum, HW), np.float32)
    u_list = []
    for i, k in enumerate(_POOLK):
        a = np.kron(_pool1d(H, k), _pool1d(W, k))
        u = np.kron(_up1d(H, H // k), _up1d(W, W // k))
        a_stack[p_bases[i]:p_bases[i] + hw_list[i]] = a
        u_list.append(u)

    n_rows = H * WP
    r_bases = np.cumsum([0] + [9 * p for p in p_list])[:-1].tolist()
    KL = 9 * psum
    l_op = np.zeros((n_rows, KL), np.float32)
    yy, xx = np.meshgrid(np.arange(H), np.arange(W), indexing="ij")
    m_valid = (yy * WP + xx).ravel()
    for t in range(9):
        dy, dx = t // 3 - 1, t % 3 - 1
        sy, sx = (yy + dy).ravel(), (xx + dx).ravel()
        ok = (sy >= 0) & (sy < H) & (sx >= 0) & (sx < W)
        src = np.where(ok, sy * W + sx, 0)
        for i, (u, hw) in enumerate(zip(u_list, hw_list)):
            col0 = r_bases[i] + t * p_list[i]
            block = np.where(ok[:, None], u[src], 0.0)
            l_op[m_valid, col0:col0 + hw] = block
    return a_stack, l_op, p_list, p_bases, r_bases


def _spp_fused_kernel(x_ref, a_ref, l_ref, w1_ref, b1_ref, wcat_ref, wx3_ref,
                      bo_ref, o_ref, xp3_ref, r_ref, *, H, W, p_list, p_bases,
                      r_bases):
    C = o_ref.shape[-1]
    WP = W + 8
    n_rows = H * WP

    x = x_ref[0].astype(jnp.bfloat16)

    pooled = jnp.dot(a_ref[...], x, preferred_element_type=jnp.float32)
    for i in range(4):
        pb, pn, rb = p_bases[i], p_list[i], r_bases[i]
        g = jnp.maximum(
            jnp.dot(pooled[pb:pb + pn].astype(jnp.bfloat16), w1_ref[i],
                    preferred_element_type=jnp.float32) + b1_ref[i], 0.0)
        res = jnp.dot(g.astype(jnp.bfloat16), wcat_ref[i],
                      preferred_element_type=jnp.float32).astype(jnp.bfloat16)
        for t in range(9):
            r_ref[rb + t * pn:rb + (t + 1) * pn, :] = res[:, t * C:(t + 1) * C]

    xp3_ref[...] = jnp.zeros(xp3_ref.shape, xp3_ref.dtype)
    for r in range(H):
        row = x[r * W:(r + 1) * W, :]
        for dj in range(3):
            xp3_ref[pl.ds((r + 1) * WP + 1 - dj, W), dj * C:(dj + 1) * C] = row

    acc = jnp.dot(l_ref[...], r_ref[...], preferred_element_type=jnp.float32)
    for di in range(3):
        acc = acc + jnp.dot(xp3_ref[pl.ds(di * WP, n_rows), :], wx3_ref[di],
                            preferred_element_type=jnp.float32)
    out = jnp.maximum(acc + bo_ref[...], 0.0)

    for r in range(H):
        o_ref[0, r * W:(r + 1) * W, :] = out[r * WP:r * WP + W, :]


def kernel(x, branch0_w, branch0_b, branch0_gamma, branch0_beta, branch0_mean,
           branch0_var, branch1_w, branch1_b, branch1_gamma, branch1_beta,
           branch1_mean, branch1_var, branch2_w, branch2_b, branch2_gamma,
           branch2_beta, branch2_mean, branch2_var, branch3_w, branch3_b,
           branch3_gamma, branch3_beta, branch3_mean, branch3_var,
           out_w, out_b, out_gamma, out_beta, out_mean, out_var):
    B, C, H, W = x.shape
    HW = H * W
    C5 = 5 * C
    WP = W + 8
    n_rows = H * WP

    a_np, l_np, p_list, p_bases, r_bases = _build_operators(H, W, WP)
    a_stack = jnp.asarray(a_np, jnp.bfloat16)
    l_op = jnp.asarray(l_np, jnp.bfloat16)
    psum = sum(p_list)

    w1_l, b1_l = [], []
    for w, b, gamma, beta, mean, var in (
            (branch0_w, branch0_b, branch0_gamma, branch0_beta, branch0_mean, branch0_var),
            (branch1_w, branch1_b, branch1_gamma, branch1_beta, branch1_mean, branch1_var),
            (branch2_w, branch2_b, branch2_gamma, branch2_beta, branch2_mean, branch2_var),
            (branch3_w, branch3_b, branch3_gamma, branch3_beta, branch3_mean, branch3_var)):
        s = gamma * jax.lax.rsqrt(var + _BN_EPS)
        w1_l.append((w * s[:, None]).T)
        b1_l.append(s * (b - mean) + beta)
    w1 = jnp.stack(w1_l).astype(jnp.bfloat16)
    b1 = jnp.stack(b1_l).reshape(4, 1, C)

    so = out_gamma * jax.lax.rsqrt(out_var + _BN_EPS)
    wf = out_w * so[:, None, None, None]
    w9 = jnp.transpose(wf, (2, 3, 1, 0)).reshape(9, C5, C)
    wcat = jnp.transpose(w9[:, :4 * C, :].reshape(9, 4, C, C),
                         (1, 2, 0, 3)).reshape(4, C, 9 * C).astype(jnp.bfloat16)
    wx3 = w9[:, 4 * C:, :].reshape(3, 3 * C, C).astype(jnp.bfloat16)
    bo = (so * (out_b - out_mean) + out_beta).reshape(1, C)

    x_cl = jnp.transpose(x.reshape(B, C, HW), (0, 2, 1))

    flops = B * 2 * (psum * HW * C + psum * C * C + 9 * psum * C * C
                     + n_rows * 9 * psum * C + 3 * n_rows * 3 * C * C)
    bytes_accessed = 4 * (B * HW * C * 2) + 2 * (n_rows * 9 * psum
                                                 + 4 * C * 9 * C + 9 * C * C)
    out_flat = pl.pallas_call(
        functools.partial(_spp_fused_kernel, H=H, W=W, p_list=p_list,
                          p_bases=p_bases, r_bases=r_bases),
        out_shape=jax.ShapeDtypeStruct((B, HW, C), jnp.float32),
        grid=(B,),
        in_specs=[
            pl.BlockSpec((1, HW, C), lambda b: (b, 0, 0)),
            pl.BlockSpec((psum, HW), lambda b: (0, 0)),
            pl.BlockSpec((n_rows, 9 * psum), lambda b: (0, 0)),
            pl.BlockSpec((4, C, C), lambda b: (0, 0, 0)),
            pl.BlockSpec((4, 1, C), lambda b: (0, 0, 0)),
            pl.BlockSpec((4, C, 9 * C), lambda b: (0, 0, 0)),
            pl.BlockSpec((3, 3 * C, C), lambda b: (0, 0, 0)),
            pl.BlockSpec((1, C), lambda b: (0, 0)),
        ],
        out_specs=pl.BlockSpec((1, HW, C), lambda b: (b, 0, 0)),
        scratch_shapes=[pltpu.VMEM(((H + 3) * WP, 3 * C), jnp.bfloat16),
                        pltpu.VMEM((9 * psum, C), jnp.bfloat16)],
        compiler_params=pltpu.CompilerParams(
            dimension_semantics=("parallel",),
            vmem_limit_bytes=48 * 1024 * 1024,
        ),
        cost_estimate=pl.CostEstimate(flops=flops, transcendentals=0,
                                      bytes_accessed=bytes_accessed),
    )(x_cl, a_stack, l_op, w1, b1, wcat, wx3, bo)

    return jnp.transpose(out_flat.reshape(B, H, W, C), (0, 3, 1, 2))
